# baseline (speedup 1.0000x reference)
import functools

import jax
import jax.numpy as jnp
from jax import lax
from jax.experimental import pallas as pl
from jax.experimental.pallas import tpu as pltpu
from jax.experimental.pallas import tpu_sc as plsc

BATCH = 16384
DIM = 128
TBL_PAD = 1048576
_NC = 2
_NS = 16
_NW = _NC * _NS
_B_PER_W = BATCH // _NW
_RG = _B_PER_W // 16
_IDR = _B_PER_W // 128
_GB = 4
_GPB = _RG // _GB


def _sc_body(f_hbm, uid_hbm, iid_hbm, tbl_hbm, w_hbm, out_hbm,
             f_v, uid_v, iid_v, bu_v, bi_v, w_v, o_v,
             sem_f, sem_ids, sem_w, sem_g):
    wid = lax.axis_index("s") * _NC + lax.axis_index("c")
    r0 = wid * _IDR

    cf = pltpu.async_copy(
        f_hbm.at[pl.ds(wid * _B_PER_W * DIM, _B_PER_W * DIM)], f_v, sem_f)
    cu = pltpu.async_copy(uid_hbm.at[pl.ds(r0, _IDR)], uid_v, sem_ids)
    ci = pltpu.async_copy(iid_hbm.at[pl.ds(r0, _IDR)], iid_v, sem_ids)
    cw = pltpu.async_copy(w_hbm, w_v, sem_w)

    cu.wait()
    ci.wait()

    for j in range(_IDR):
        for c in range(8):
            s = pl.ds(c * 16, 16)
            iid_v[j, s] = iid_v[j, s] + TBL_PAD

    gathers = []
    for j in range(_IDR):
        gathers.append(pltpu.async_copy(tbl_hbm.at[uid_v.at[j]], bu_v.at[j], sem_g))
        gathers.append(pltpu.async_copy(tbl_hbm.at[iid_v.at[j]], bi_v.at[j], sem_g))

    cw.wait()
    cf.wait()

    lane = jax.lax.iota(jnp.int32, 16)
    lane_base = lane * DIM
    bias = w_v[pl.ds(DIM * 16, 16)]

    for gb in range(_GB):
        bases = [lane_base + (gb * _GPB + g) * 16 * DIM for g in range(_GPB)]

        def mv_body(i, accs, bases=bases):
            accs = list(accs)
            for u in range(4):
                d = i * 4 + u
                wd = plsc.load_gather(w_v, [d * 16 + lane])
                for g in range(_GPB):
                    fv = plsc.load_gather(f_v, [bases[g] + d])
                    accs[g] = accs[g] + fv * wd
            return tuple(accs)

        accs = lax.fori_loop(0, DIM // 4, mv_body,
                             tuple(bias for _ in range(_GPB)))
        for g in range(_GPB):
            o_v[pl.ds((gb * _GPB + g) * 16, 16)] = accs[g]

    for c in gathers:
        c.wait()

    for j in range(_IDR):
        for c in range(8):
            s16 = pl.ds(c * 16, 16)
            s = pl.ds(j * 128 + c * 16, 16)
            o_v[s] = o_v[s] + bu_v[j, s16] + bi_v[j, s16]

    pltpu.sync_copy(o_v, out_hbm.at[pl.ds(wid * _B_PER_W, _B_PER_W)])


_sc_lfm = functools.partial(
    pl.kernel,
    out_type=jax.ShapeDtypeStruct((BATCH,), jnp.float32),
    mesh=plsc.VectorSubcoreMesh(core_axis_name="c", subcore_axis_name="s",
                                num_cores=_NC, num_subcores=_NS),
    compiler_params=pltpu.CompilerParams(needs_layout_passes=False),
    scratch_types=[
        pltpu.VMEM((_B_PER_W * DIM,), jnp.float32),
        pltpu.VMEM((_IDR, 128), jnp.int32),
        pltpu.VMEM((_IDR, 128), jnp.int32),
        pltpu.VMEM((_IDR, 128), jnp.float32),
        pltpu.VMEM((_IDR, 128), jnp.float32),
        pltpu.VMEM((DIM * 16 + 16,), jnp.float32),
        pltpu.VMEM((_B_PER_W,), jnp.float32),
        pltpu.SemaphoreType.DMA,
        pltpu.SemaphoreType.DMA,
        pltpu.SemaphoreType.DMA,
        pltpu.SemaphoreType.DMA,
    ],
)(_sc_body)


def kernel(feature, user_id, item_id, fc_w, fc_b, b_users, b_items):
    f_flat = feature.reshape(-1)
    uid2 = user_id.reshape(BATCH // 128, 128)
    iid2 = item_id.reshape(BATCH // 128, 128)
    zpad = jnp.zeros((TBL_PAD - b_users.shape[0], 1), jnp.float32)
    tbl = jnp.concatenate([b_users, zpad, b_items, zpad]).reshape(-1)
    wtab = jnp.concatenate([
        jnp.repeat(fc_w[0], 16),
        jnp.repeat(fc_b, 16),
    ])
    return _sc_lfm(f_flat, uid2, iid2, tbl, wtab)


# speedup vs baseline: 2.6344x; 2.6344x over previous
import functools

import jax
import jax.numpy as jnp
from jax import lax
from jax.experimental import pallas as pl
from jax.experimental.pallas import tpu as pltpu
from jax.experimental.pallas import tpu_sc as plsc

BATCH = 16384
DIM = 128
TBL_PAD = 1048576
_NC = 2
_NS = 16
_NW = _NC * _NS
_B_PER_W = BATCH // _NW
_RG = _B_PER_W // 16
_IDR = _B_PER_W // 128
_GB = 4
_GPB = _RG // _GB


def _sc_body(f_hbm, uid_hbm, iid_hbm, bu_hbm, bi_hbm, w_hbm, out_hbm,
             f_v, uid_v, iid_v, bu_v, bi_v, w_v, o_v, fidx_v,
             sem_f, sem_ids, sem_w, sem_g):
    wid = lax.axis_index("s") * _NC + lax.axis_index("c")
    r0 = wid * _IDR
    lane = jax.lax.iota(jnp.int32, 16)

    cu = pltpu.async_copy(uid_hbm.at[pl.ds(r0, _IDR)], uid_v, sem_ids)
    ci = pltpu.async_copy(iid_hbm.at[pl.ds(r0, _IDR)], iid_v, sem_ids)
    cw = pltpu.async_copy(w_hbm, w_v, sem_w)

    fbase = wid * _B_PER_W
    for j in range(_IDR):
        for c in range(8):
            fidx_v[j, pl.ds(c * 16, 16)] = lane + (fbase + j * 128 + c * 16)
    fcopies = [
        pltpu.async_copy(f_hbm.at[fidx_v.at[j]],
                         f_v.at[pl.ds(j * 128, 128)], sem_f)
        for j in range(_IDR)
    ]

    cu.wait()
    ci.wait()

    gathers = []
    for j in range(_IDR):
        gathers.append(pltpu.async_copy(bu_hbm.at[uid_v.at[j]], bu_v.at[j], sem_g))
        gathers.append(pltpu.async_copy(bi_hbm.at[iid_v.at[j]], bi_v.at[j], sem_g))

    cw.wait()
    for c in fcopies:
        c.wait()
    bias = w_v[16, pl.ds(0, 16)]

    for gb in range(_GB):
        rows = [lane + (gb * _GPB + g) * 16 for g in range(_GPB)]

        def mv_body(i, accs, rows=rows):
            accs = list(accs)
            for u in range(4):
                d = i * 4 + u
                drow = jnp.full((16,), lax.shift_right_logical(d, 3), jnp.int32)
                dcol = jnp.bitwise_and(d, 7) * 16 + lane
                wd = plsc.load_gather(w_v, [drow, dcol])
                dsplat = jnp.full((16,), d, jnp.int32)
                for g in range(_GPB):
                    fv = plsc.load_gather(f_v, [rows[g], dsplat])
                    accs[g] = accs[g] + fv * wd
            return tuple(accs)

        accs = lax.fori_loop(0, DIM // 4, mv_body,
                             tuple(bias for _ in range(_GPB)))
        for g in range(_GPB):
            o_v[pl.ds((gb * _GPB + g) * 16, 16)] = accs[g]

    for c in gathers:
        c.wait()

    for j in range(_IDR):
        for c in range(8):
            s16 = pl.ds(c * 16, 16)
            s = pl.ds(j * 128 + c * 16, 16)
            o_v[s] = o_v[s] + bu_v[j, s16] + bi_v[j, s16]

    pltpu.sync_copy(o_v, out_hbm.at[pl.ds(wid * _B_PER_W, _B_PER_W)])


_sc_lfm = functools.partial(
    pl.kernel,
    out_type=jax.ShapeDtypeStruct((BATCH,), jnp.float32),
    mesh=plsc.VectorSubcoreMesh(core_axis_name="c", subcore_axis_name="s",
                                num_cores=_NC, num_subcores=_NS),
    compiler_params=pltpu.CompilerParams(needs_layout_passes=False),
    scratch_types=[
        pltpu.VMEM((_B_PER_W, DIM), jnp.float32),
        pltpu.VMEM((_IDR, 128), jnp.int32),
        pltpu.VMEM((_IDR, 128), jnp.int32),
        pltpu.VMEM((_IDR, 128), jnp.float32),
        pltpu.VMEM((_IDR, 128), jnp.float32),
        pltpu.VMEM((17, 128), jnp.float32),
        pltpu.VMEM((_B_PER_W,), jnp.float32),
        pltpu.VMEM((_IDR, 128), jnp.int32),
        pltpu.SemaphoreType.DMA,
        pltpu.SemaphoreType.DMA,
        pltpu.SemaphoreType.DMA,
        pltpu.SemaphoreType.DMA,
    ],
)(_sc_body)


def kernel(feature, user_id, item_id, fc_w, fc_b, b_users, b_items):
    uid2 = user_id.reshape(BATCH // 128, 128)
    iid2 = item_id.reshape(BATCH // 128, 128)
    bu_flat = jnp.pad(b_users, ((0, TBL_PAD - b_users.shape[0]), (0, 0))).reshape(-1)
    bi_flat = jnp.pad(b_items, ((0, TBL_PAD - b_items.shape[0]), (0, 0))).reshape(-1)
    wtab = jnp.concatenate([
        jnp.repeat(fc_w[0], 16),
        jnp.repeat(fc_b, 16),
        jnp.zeros((112,), jnp.float32),
    ]).reshape(17, 128)
    return _sc_lfm(feature, uid2, iid2, bu_flat, bi_flat, wtab)


# speedup vs baseline: 3.8211x; 1.4505x over previous
import functools

import jax
import jax.numpy as jnp
from jax import lax
from jax.experimental import pallas as pl
from jax.experimental.pallas import tpu as pltpu
from jax.experimental.pallas import tpu_sc as plsc

BATCH = 16384
DIM = 128
TBL_PAD = 1048576
_ROWS = BATCH // 128
_NC = 2
_NS = 16
_NW = _NC * _NS
_B_PER_W = BATCH // _NW
_IDR = _B_PER_W // 128


def _sc_body(uid_hbm, iid_hbm, bu_hbm, bi_hbm, out_hbm,
             uid_v, iid_v, bu_v, bi_v, o_v, sem_ids, sem_g):
    wid = lax.axis_index("s") * _NC + lax.axis_index("c")
    r0 = wid * _IDR

    cu = pltpu.async_copy(uid_hbm.at[pl.ds(r0, _IDR)], uid_v, sem_ids)
    ci = pltpu.async_copy(iid_hbm.at[pl.ds(r0, _IDR)], iid_v, sem_ids)
    cu.wait()
    ci.wait()

    gathers = []
    for j in range(_IDR):
        gathers.append(pltpu.async_copy(bu_hbm.at[uid_v.at[j]], bu_v.at[j], sem_g))
        gathers.append(pltpu.async_copy(bi_hbm.at[iid_v.at[j]], bi_v.at[j], sem_g))
    for c in gathers:
        c.wait()

    for j in range(_IDR):
        for c in range(8):
            s16 = pl.ds(c * 16, 16)
            o_v[j, s16] = bu_v[j, s16] + bi_v[j, s16]

    pltpu.sync_copy(o_v, out_hbm.at[pl.ds(r0, _IDR)])


_sc_gather = functools.partial(
    pl.kernel,
    out_type=jax.ShapeDtypeStruct((_ROWS, 128), jnp.float32),
    mesh=plsc.VectorSubcoreMesh(core_axis_name="c", subcore_axis_name="s",
                                num_cores=_NC, num_subcores=_NS),
    scratch_types=[
        pltpu.VMEM((_IDR, 128), jnp.int32),
        pltpu.VMEM((_IDR, 128), jnp.int32),
        pltpu.VMEM((_IDR, 128), jnp.float32),
        pltpu.VMEM((_IDR, 128), jnp.float32),
        pltpu.VMEM((_IDR, 128), jnp.float32),
        pltpu.SemaphoreType.DMA,
        pltpu.SemaphoreType.DMA,
    ],
)(_sc_body)


def _tc_body(f_ref, w_ref, b_ref, g_ref, o_ref):
    acc = jnp.sum(f_ref[...] * w_ref[...], axis=2)
    o_ref[...] = acc + g_ref[...] + b_ref[0, 0]


def _tc_matvec_add(f3, fc_w3, fc_b2, g2):
    grid = (16,)
    rb = _ROWS // grid[0]
    return pl.pallas_call(
        _tc_body,
        grid=grid,
        in_specs=[
            pl.BlockSpec((rb, 128, DIM), lambda i: (i, 0, 0)),
            pl.BlockSpec((1, 1, DIM), lambda i: (0, 0, 0)),
            pl.BlockSpec(memory_space=pltpu.SMEM),
            pl.BlockSpec((rb, 128), lambda i: (i, 0)),
        ],
        out_specs=pl.BlockSpec((rb, 128), lambda i: (i, 0)),
        out_shape=jax.ShapeDtypeStruct((_ROWS, 128), jnp.float32),
    )(f3, fc_w3, fc_b2, g2)


def kernel(feature, user_id, item_id, fc_w, fc_b, b_users, b_items):
    uid2 = user_id.reshape(_ROWS, 128)
    iid2 = item_id.reshape(_ROWS, 128)
    bu_flat = jnp.pad(b_users, ((0, TBL_PAD - b_users.shape[0]), (0, 0))).reshape(-1)
    bi_flat = jnp.pad(b_items, ((0, TBL_PAD - b_items.shape[0]), (0, 0))).reshape(-1)
    g2 = _sc_gather(uid2, iid2, bu_flat, bi_flat)
    f3 = feature.reshape(_ROWS, 128, DIM)
    out2 = _tc_matvec_add(f3, fc_w.reshape(1, 1, DIM), fc_b.reshape(1, 1), g2)
    return out2.reshape(BATCH)


# speedup vs baseline: 4.1322x; 1.0814x over previous
import functools

import jax
import jax.numpy as jnp
from jax import lax
from jax.experimental import pallas as pl
from jax.experimental.pallas import tpu as pltpu
from jax.experimental.pallas import tpu_sc as plsc

BATCH = 16384
DIM = 128
TBL_PAD = 1048576
_ROWS = BATCH // 128
_NC = 2
_NS = 16
_NW = _NC * _NS
_B_PER_W = BATCH // _NW
_IDR = _B_PER_W // 128


def _sc_body(uid_hbm, iid_hbm, bu_hbm, bi_hbm, out_hbm,
             uid_v, iid_v, bu_v, bi_v, o_v, sem_ids, sem_g):
    wid = lax.axis_index("s") * _NC + lax.axis_index("c")
    r0 = wid * _IDR

    cu = pltpu.async_copy(uid_hbm.at[pl.ds(r0, _IDR)], uid_v, sem_ids)
    ci = pltpu.async_copy(iid_hbm.at[pl.ds(r0, _IDR)], iid_v, sem_ids)
    cu.wait()
    ci.wait()

    gathers = []
    for j in range(_IDR):
        gathers.append(pltpu.async_copy(bu_hbm.at[uid_v.at[j]], bu_v.at[j], sem_g))
        gathers.append(pltpu.async_copy(bi_hbm.at[iid_v.at[j]], bi_v.at[j], sem_g))
    for c in gathers:
        c.wait()

    for j in range(_IDR):
        for c in range(8):
            s16 = pl.ds(c * 16, 16)
            o_v[j, s16] = bu_v[j, s16] + bi_v[j, s16]

    pltpu.sync_copy(o_v, out_hbm.at[pl.ds(r0, _IDR)])


_sc_gather = functools.partial(
    pl.kernel,
    out_type=jax.ShapeDtypeStruct((_ROWS, 128), jnp.float32),
    mesh=plsc.VectorSubcoreMesh(core_axis_name="c", subcore_axis_name="s",
                                num_cores=_NC, num_subcores=_NS),
    scratch_types=[
        pltpu.VMEM((_IDR, 128), jnp.int32),
        pltpu.VMEM((_IDR, 128), jnp.int32),
        pltpu.VMEM((_IDR, 128), jnp.float32),
        pltpu.VMEM((_IDR, 128), jnp.float32),
        pltpu.VMEM((_IDR, 128), jnp.float32),
        pltpu.SemaphoreType.DMA,
        pltpu.SemaphoreType.DMA,
    ],
)(_sc_body)


def _tc_body(f_ref, w_ref, b_ref, o_ref):
    acc = jnp.sum(f_ref[...] * w_ref[...], axis=2)
    o_ref[...] = acc + b_ref[0, 0]


def _tc_matvec(f3, fc_w3, fc_b2):
    grid = (16,)
    rb = _ROWS // grid[0]
    return pl.pallas_call(
        _tc_body,
        grid=grid,
        in_specs=[
            pl.BlockSpec((rb, 128, DIM), lambda i: (i, 0, 0)),
            pl.BlockSpec((1, 1, DIM), lambda i: (0, 0, 0)),
            pl.BlockSpec(memory_space=pltpu.SMEM),
        ],
        out_specs=pl.BlockSpec((rb, 128), lambda i: (i, 0)),
        out_shape=jax.ShapeDtypeStruct((_ROWS, 128), jnp.float32),
    )(f3, fc_w3, fc_b2)


def _tc_add_body(a_ref, b_ref, o_ref):
    o_ref[...] = a_ref[...] + b_ref[...]


def _tc_add(a2, b2):
    return pl.pallas_call(
        _tc_add_body,
        out_shape=jax.ShapeDtypeStruct((_ROWS, 128), jnp.float32),
    )(a2, b2)


def kernel(feature, user_id, item_id, fc_w, fc_b, b_users, b_items):
    uid2 = user_id.reshape(_ROWS, 128)
    iid2 = item_id.reshape(_ROWS, 128)
    bu_flat = jnp.pad(b_users, ((0, TBL_PAD - b_users.shape[0]), (0, 0))).reshape(-1)
    bi_flat = jnp.pad(b_items, ((0, TBL_PAD - b_items.shape[0]), (0, 0))).reshape(-1)
    g2 = _sc_gather(uid2, iid2, bu_flat, bi_flat)
    f3 = feature.reshape(_ROWS, 128, DIM)
    fc2 = _tc_matvec(f3, fc_w.reshape(1, 1, DIM), fc_b.reshape(1, 1))
    out2 = _tc_add(fc2, g2)
    return out2.reshape(BATCH)


# speedup vs baseline: 4.5023x; 1.0896x over previous
import functools

import jax
import jax.numpy as jnp
from jax import lax
from jax.experimental import pallas as pl
from jax.experimental.pallas import tpu as pltpu
from jax.experimental.pallas import tpu_sc as plsc

BATCH = 16384
DIM = 128
TBL_PAD = 1048576
_ROWS = BATCH // 128
_NC = 2
_NS = 16
_NW = _NC * _NS
_B_PER_W = BATCH // _NW
_IDR = _B_PER_W // 128


def _sc_body(uid_hbm, iid_hbm, bu_hbm, bi_hbm, out_hbm,
             uid_v, iid_v, bu_v, bi_v, o_v, sem_ids, sem_g):
    wid = lax.axis_index("s") * _NC + lax.axis_index("c")
    r0 = wid * _IDR

    cu = pltpu.async_copy(uid_hbm.at[pl.ds(r0, _IDR)], uid_v, sem_ids)
    ci = pltpu.async_copy(iid_hbm.at[pl.ds(r0, _IDR)], iid_v, sem_ids)
    cu.wait()
    ci.wait()

    gathers = []
    for j in range(_IDR):
        gathers.append(pltpu.async_copy(bu_hbm.at[uid_v.at[j]], bu_v.at[j], sem_g))
        gathers.append(pltpu.async_copy(bi_hbm.at[iid_v.at[j]], bi_v.at[j], sem_g))
    for c in gathers:
        c.wait()

    for j in range(_IDR):
        for c in range(8):
            s16 = pl.ds(c * 16, 16)
            o_v[j, s16] = bu_v[j, s16] + bi_v[j, s16]

    pltpu.sync_copy(o_v, out_hbm.at[pl.ds(r0, _IDR)])


_sc_gather = functools.partial(
    pl.kernel,
    out_type=jax.ShapeDtypeStruct((_ROWS, 128), jnp.float32),
    mesh=plsc.VectorSubcoreMesh(core_axis_name="c", subcore_axis_name="s",
                                num_cores=_NC, num_subcores=_NS),
    scratch_types=[
        pltpu.VMEM((_IDR, 128), jnp.int32),
        pltpu.VMEM((_IDR, 128), jnp.int32),
        pltpu.VMEM((_IDR, 128), jnp.float32),
        pltpu.VMEM((_IDR, 128), jnp.float32),
        pltpu.VMEM((_IDR, 128), jnp.float32),
        pltpu.SemaphoreType.DMA,
        pltpu.SemaphoreType.DMA,
    ],
)(_sc_body)


def _tc_body(f_ref, w_ref, b_ref, o_ref):
    acc = jnp.sum(f_ref[...] * w_ref[...], axis=2)
    o_ref[...] = acc + b_ref[0, 0]


def _tc_matvec(f3, fc_w3, fc_b2):
    grid = (8,)
    rb = _ROWS // grid[0]
    return pl.pallas_call(
        _tc_body,
        grid=grid,
        in_specs=[
            pl.BlockSpec((rb, 128, DIM), lambda i: (i, 0, 0)),
            pl.BlockSpec((1, 1, DIM), lambda i: (0, 0, 0)),
            pl.BlockSpec(memory_space=pltpu.SMEM),
        ],
        out_specs=pl.BlockSpec((rb, 128), lambda i: (i, 0)),
        out_shape=jax.ShapeDtypeStruct((_ROWS, 128), jnp.float32),
    )(f3, fc_w3, fc_b2)


def _tc_add_body(a_ref, b_ref, o_ref):
    o_ref[...] = a_ref[...] + b_ref[...]


def _tc_add(a2, b2):
    return pl.pallas_call(
        _tc_add_body,
        out_shape=jax.ShapeDtypeStruct((_ROWS, 128), jnp.float32),
    )(a2, b2)


def kernel(feature, user_id, item_id, fc_w, fc_b, b_users, b_items):
    uid2 = user_id.reshape(_ROWS, 128)
    iid2 = item_id.reshape(_ROWS, 128)
    bu_flat = jnp.pad(b_users, ((0, TBL_PAD - b_users.shape[0]), (0, 0))).reshape(-1)
    bi_flat = jnp.pad(b_items, ((0, TBL_PAD - b_items.shape[0]), (0, 0))).reshape(-1)
    g2 = _sc_gather(uid2, iid2, bu_flat, bi_flat)
    f3 = feature.reshape(_ROWS, 128, DIM)
    fc2 = _tc_matvec(f3, fc_w.reshape(1, 1, DIM), fc_b.reshape(1, 1))
    out2 = _tc_add(fc2, g2)
    return out2.reshape(BATCH)


# speedup vs baseline: 4.6855x; 1.0407x over previous
import functools

import jax
import jax.numpy as jnp
from jax import lax
from jax.experimental import pallas as pl
from jax.experimental.pallas import tpu as pltpu
from jax.experimental.pallas import tpu_sc as plsc

BATCH = 16384
DIM = 128
TBL_PAD = 1048576
_ROWS = BATCH // 128
_NC = 2
_NS = 16
_NW = _NC * _NS
_B_PER_W = BATCH // _NW
_IDR = _B_PER_W // 128


def _sc_body(uid_hbm, iid_hbm, bu_hbm, bi_hbm, out_hbm,
             uid_v, iid_v, bu_v, bi_v, o_v, sem_ids, sem_g):
    wid = lax.axis_index("s") * _NC + lax.axis_index("c")
    r0 = wid * _IDR

    cu = pltpu.async_copy(uid_hbm.at[pl.ds(r0, _IDR)], uid_v, sem_ids)
    ci = pltpu.async_copy(iid_hbm.at[pl.ds(r0, _IDR)], iid_v, sem_ids)
    cu.wait()
    ci.wait()

    gathers = []
    for j in range(_IDR):
        gathers.append(pltpu.async_copy(bu_hbm.at[uid_v.at[j]], bu_v.at[j], sem_g))
        gathers.append(pltpu.async_copy(bi_hbm.at[iid_v.at[j]], bi_v.at[j], sem_g))
    for c in gathers:
        c.wait()

    for j in range(_IDR):
        for c in range(8):
            s16 = pl.ds(c * 16, 16)
            o_v[j, s16] = bu_v[j, s16] + bi_v[j, s16]

    pltpu.sync_copy(o_v, out_hbm.at[pl.ds(r0, _IDR)])


_sc_gather = functools.partial(
    pl.kernel,
    out_type=jax.ShapeDtypeStruct((_ROWS, 128), jnp.float32),
    mesh=plsc.VectorSubcoreMesh(core_axis_name="c", subcore_axis_name="s",
                                num_cores=_NC, num_subcores=_NS),
    scratch_types=[
        pltpu.VMEM((_IDR, 128), jnp.int32),
        pltpu.VMEM((_IDR, 128), jnp.int32),
        pltpu.VMEM((_IDR, 128), jnp.float32),
        pltpu.VMEM((_IDR, 128), jnp.float32),
        pltpu.VMEM((_IDR, 128), jnp.float32),
        pltpu.SemaphoreType.DMA,
        pltpu.SemaphoreType.DMA,
    ],
)(_sc_body)


def _tc_body(f_ref, w_ref, b_ref, o_ref):
    acc = jnp.sum(f_ref[...] * w_ref[...], axis=2)
    o_ref[...] = acc + b_ref[0, 0]


def _tc_matvec(f3, fc_w3, fc_b2):
    grid = (4,)
    rb = _ROWS // grid[0]
    return pl.pallas_call(
        _tc_body,
        grid=grid,
        in_specs=[
            pl.BlockSpec((rb, 128, DIM), lambda i: (i, 0, 0)),
            pl.BlockSpec((1, 1, DIM), lambda i: (0, 0, 0)),
            pl.BlockSpec(memory_space=pltpu.SMEM),
        ],
        out_specs=pl.BlockSpec((rb, 128), lambda i: (i, 0)),
        out_shape=jax.ShapeDtypeStruct((_ROWS, 128), jnp.float32),
    )(f3, fc_w3, fc_b2)


def _tc_add_body(a_ref, b_ref, o_ref):
    o_ref[...] = a_ref[...] + b_ref[...]


def _tc_add(a2, b2):
    return pl.pallas_call(
        _tc_add_body,
        out_shape=jax.ShapeDtypeStruct((_ROWS, 128), jnp.float32),
    )(a2, b2)


def kernel(feature, user_id, item_id, fc_w, fc_b, b_users, b_items):
    uid2 = user_id.reshape(_ROWS, 128)
    iid2 = item_id.reshape(_ROWS, 128)
    bu_flat = jnp.pad(b_users, ((0, TBL_PAD - b_users.shape[0]), (0, 0))).reshape(-1)
    bi_flat = jnp.pad(b_items, ((0, TBL_PAD - b_items.shape[0]), (0, 0))).reshape(-1)
    g2 = _sc_gather(uid2, iid2, bu_flat, bi_flat)
    f3 = feature.reshape(_ROWS, 128, DIM)
    fc2 = _tc_matvec(f3, fc_w.reshape(1, 1, DIM), fc_b.reshape(1, 1))
    out2 = _tc_add(fc2, g2)
    return out2.reshape(BATCH)
